# Initial kernel scaffold; baseline (speedup 1.0000x reference)
import jax
import jax.numpy as jnp
from jax.experimental import pallas as pl


def kernel(x, edge_index, edge_weight, W1, b1, W2, b2):
    raise NotImplementedError("write your pallas kernel here")


# speedup vs baseline: 7.0838x; 7.0838x over previous
import functools

import jax
import jax.numpy as jnp
from jax import lax
from jax.experimental import pallas as pl
from jax.experimental.pallas import tpu as pltpu
from jax.experimental.pallas import tpu_sc as plsc

NC = 2
NS = 16
NW = NC * NS
CHUNK = 128
LANES = 16

_MESH = plsc.VectorSubcoreMesh(core_axis_name="c", subcore_axis_name="s")


def _make_deg_kernel(npad, nchunk):
    rows_per_tile = npad // NS

    @functools.partial(
        pl.kernel,
        out_type=jax.ShapeDtypeStruct((NC, npad), jnp.float32),
        mesh=_MESH,
        scratch_types=[
            pltpu.VMEM((nchunk, CHUNK), jnp.int32),
            pltpu.VMEM((nchunk, CHUNK), jnp.float32),
            pltpu.VMEM((rows_per_tile,), jnp.float32),
            pltpu.VMEM_SHARED((npad,), jnp.float32),
        ],
    )
    def deg_kernel(dst_hbm, ew_hbm, out_hbm, dst_v, ew_v, zbuf, deg_s):
        c = lax.axis_index("c")
        s = lax.axis_index("s")
        wid = c * NS + s
        for i in range(rows_per_tile // LANES):
            zbuf[pl.ds(i * LANES, LANES)] = jnp.zeros((LANES,), jnp.float32)
        pltpu.sync_copy(zbuf, deg_s.at[pl.ds(s * rows_per_tile, rows_per_tile)])
        plsc.subcore_barrier()
        pltpu.sync_copy(dst_hbm.at[wid], dst_v)
        pltpu.sync_copy(ew_hbm.at[wid], ew_v)

        def chunk_body(j, carry):
            pltpu.sync_copy(ew_v.at[j], deg_s.at[dst_v.at[j]], add=True)
            return carry

        lax.fori_loop(0, nchunk, chunk_body, 0)
        plsc.subcore_barrier()
        pltpu.sync_copy(deg_s.at[pl.ds(s * rows_per_tile, rows_per_tile)],
                        out_hbm.at[c, pl.ds(s * rows_per_tile, rows_per_tile)])

    return deg_kernel


def _make_agg_kernel(npad, nchunk, dh):
    rows_per_tile = npad // NS

    @functools.partial(
        pl.kernel,
        out_type=jax.ShapeDtypeStruct((NC * npad, dh), jnp.float32),
        mesh=_MESH,
        scratch_types=[
            pltpu.VMEM((nchunk, CHUNK), jnp.int32),
            pltpu.VMEM((nchunk, CHUNK), jnp.int32),
            pltpu.VMEM((nchunk, CHUNK), jnp.float32),
            pltpu.VMEM((CHUNK, dh), jnp.float32),
            pltpu.VMEM_SHARED((npad, dh), jnp.float32),
        ],
    )
    def agg_kernel(y_hbm, src_hbm, dst_hbm, ew_hbm, out_hbm,
                   src_v, dst_v, ew_v, rows_v, accum_s):
        c = lax.axis_index("c")
        s = lax.axis_index("s")
        r0 = s * rows_per_tile
        pltpu.sync_copy(y_hbm.at[pl.ds(c * npad + r0, rows_per_tile)],
                        accum_s.at[pl.ds(r0, rows_per_tile)])
        plsc.subcore_barrier()
        pltpu.sync_copy(src_hbm.at[c, s], src_v)
        pltpu.sync_copy(dst_hbm.at[s], dst_v)
        pltpu.sync_copy(ew_hbm.at[s], ew_v)

        def chunk_body(j, carry):
            pltpu.sync_copy(y_hbm.at[src_v.at[j]], rows_v)

            def group_body(g, c2):
                ew16 = ew_v[j, pl.ds(g * LANES, LANES)]
                for l in range(LANES):
                    w = ew16[l]
                    e = g * LANES + l
                    for d in range(dh // LANES):
                        sl = pl.ds(d * LANES, LANES)
                        rows_v[e, sl] = rows_v[e, sl] * w
                return c2

            lax.fori_loop(0, CHUNK // LANES, group_body, 0)
            pltpu.sync_copy(rows_v, accum_s.at[dst_v.at[j]], add=True)
            return carry

        lax.fori_loop(0, nchunk, chunk_body, 0)
        plsc.subcore_barrier()
        pltpu.sync_copy(accum_s.at[pl.ds(r0, rows_per_tile)],
                        out_hbm.at[pl.ds(c * npad + r0, rows_per_tile)])

    return agg_kernel


def _make_agg_split_kernel(npad, nchunk, dh):
    rows_per_tile = npad // NS

    @functools.partial(
        pl.kernel,
        out_type=jax.ShapeDtypeStruct((NC * npad, dh), jnp.float32),
        mesh=_MESH,
        scratch_types=[
            pltpu.VMEM((nchunk, CHUNK), jnp.int32),
            pltpu.VMEM((nchunk, CHUNK), jnp.int32),
            pltpu.VMEM((nchunk, CHUNK), jnp.float32),
            pltpu.VMEM((CHUNK, dh), jnp.float32),
            pltpu.VMEM_SHARED((npad, dh), jnp.float32),
        ],
    )
    def agg_kernel(y_hbm, src_hbm, dst_hbm, ew_hbm, out_hbm,
                   src_v, dst_v, ew_v, rows_v, accum_s):
        c = lax.axis_index("c")
        s = lax.axis_index("s")
        wid = c * NS + s
        r0 = s * rows_per_tile
        for d in range(dh // LANES):
            z = jnp.zeros((LANES,), jnp.float32)
            for i in range(CHUNK):
                rows_v[i, pl.ds(d * LANES, LANES)] = z
        for b in range(rows_per_tile // CHUNK):
            pltpu.sync_copy(rows_v, accum_s.at[pl.ds(r0 + b * CHUNK, CHUNK)])
        plsc.subcore_barrier()
        pltpu.sync_copy(src_hbm.at[wid], src_v)
        pltpu.sync_copy(dst_hbm.at[wid], dst_v)
        pltpu.sync_copy(ew_hbm.at[wid], ew_v)

        def chunk_body(j, carry):
            pltpu.sync_copy(y_hbm.at[src_v.at[j]], rows_v)

            def group_body(g, c2):
                ew16 = ew_v[j, pl.ds(g * LANES, LANES)]
                for l in range(LANES):
                    w = ew16[l]
                    e = g * LANES + l
                    for d in range(dh // LANES):
                        sl = pl.ds(d * LANES, LANES)
                        rows_v[e, sl] = rows_v[e, sl] * w
                return c2

            lax.fori_loop(0, CHUNK // LANES, group_body, 0)
            pltpu.sync_copy(rows_v, accum_s.at[dst_v.at[j]], add=True)
            return carry

        lax.fori_loop(0, nchunk, chunk_body, 0)
        plsc.subcore_barrier()
        pltpu.sync_copy(accum_s.at[pl.ds(r0, rows_per_tile)],
                        out_hbm.at[pl.ds(c * npad + r0, rows_per_tile)])

    return agg_kernel


def _k1_body(x_ref, w_ref, deg_ref, y_ref, dis_ref):
    deg = deg_ref[0, :] + deg_ref[1, :] + 1.0
    dis = lax.rsqrt(deg)
    y = jnp.dot(x_ref[...], w_ref[...], preferred_element_type=jnp.float32,
                precision=lax.Precision.HIGHEST)
    y = y * dis[:, None]
    half = y.shape[1] // 2
    y_ref[0] = y[:, :half]
    y_ref[1] = y[:, half:]
    dis_ref[...] = dis[:, None]


def _k3_body(agg_ref, dis_ref, b_ref, w_ref, y_ref):
    dis = dis_ref[...]
    h = jnp.concatenate([agg_ref[0], agg_ref[1]], axis=-1)
    h = jnp.maximum(h * dis + b_ref[...], 0.0)
    y = jnp.dot(h, w_ref[...], preferred_element_type=jnp.float32,
                precision=lax.Precision.HIGHEST)
    y_ref[...] = y * dis


def _k5_body(agg_ref, y2_ref, dis_ref, b_ref, out_ref):
    o = agg_ref[0] + agg_ref[1] + y2_ref[...]
    out_ref[...] = o * dis_ref[...] + b_ref[...]


def _ceil_to(v, m):
    return -(-v // m) * m


def kernel(x, edge_index, edge_weight, W1, b1, W2, b2):
    n, d_in = x.shape
    d_mid = W1.shape[1]
    d_out = W2.shape[1]
    e = edge_index.shape[1]

    src = edge_index[0].astype(jnp.int32)
    dst = edge_index[1].astype(jnp.int32)
    ew = edge_weight.astype(jnp.float32)

    npad = _ceil_to(n, 1024)
    e2 = _ceil_to(e, CHUNK * NW)
    pad = e2 - e
    src = jnp.pad(src, (0, pad))
    dst = jnp.pad(dst, (0, pad))
    ew = jnp.pad(ew, (0, pad))

    dst_w = dst.reshape(NW, -1, CHUNK)
    ew_w = ew.reshape(NW, -1, CHUNK)
    nchunk_w = dst_w.shape[1]

    src_s = src.reshape(NS, -1, CHUNK)
    dst_s = dst.reshape(NS, -1, CHUNK)
    ew_s = ew.reshape(NS, -1, CHUNK)
    nchunk_s = src_s.shape[1]
    src_off = jnp.stack([src_s, src_s + npad], axis=0)

    xp = jnp.pad(x, ((0, npad - n), (0, 0)))

    deg_parts = _make_deg_kernel(npad, nchunk_w)(dst_w, ew_w)

    grid = npad // 1024
    hm = d_mid // 2
    ho = d_out // 2

    y1, dis = pl.pallas_call(
        _k1_body,
        grid=(grid,),
        in_specs=[
            pl.BlockSpec((1024, d_in), lambda r: (r, 0)),
            pl.BlockSpec((d_in, d_mid), lambda r: (0, 0)),
            pl.BlockSpec((NC, 1024), lambda r: (0, r)),
        ],
        out_specs=[
            pl.BlockSpec((NC, 1024, hm), lambda r: (0, r, 0)),
            pl.BlockSpec((1024, 1), lambda r: (r, 0)),
        ],
        out_shape=[
            jax.ShapeDtypeStruct((NC, npad, hm), jnp.float32),
            jax.ShapeDtypeStruct((npad, 1), jnp.float32),
        ],
    )(xp, W1, deg_parts)

    agg1 = _make_agg_kernel(npad, nchunk_s, hm)(
        y1.reshape(NC * npad, hm), src_off, dst_s, ew_s)
    agg1 = agg1.reshape(NC, npad, hm)

    y2 = pl.pallas_call(
        _k3_body,
        grid=(grid,),
        in_specs=[
            pl.BlockSpec((NC, 1024, hm), lambda r: (0, r, 0)),
            pl.BlockSpec((1024, 1), lambda r: (r, 0)),
            pl.BlockSpec((1, d_mid), lambda r: (0, 0)),
            pl.BlockSpec((d_mid, d_out), lambda r: (0, 0)),
        ],
        out_specs=pl.BlockSpec((1024, d_out), lambda r: (r, 0)),
        out_shape=jax.ShapeDtypeStruct((npad, d_out), jnp.float32),
    )(agg1, dis, b1.reshape(1, d_mid), W2)

    agg2 = _make_agg_split_kernel(npad, nchunk_w, d_out)(
        y2, src.reshape(NW, -1, CHUNK), dst_w, ew_w)
    agg2 = agg2.reshape(NC, npad, d_out)

    out = pl.pallas_call(
        _k5_body,
        grid=(grid,),
        in_specs=[
            pl.BlockSpec((NC, 1024, d_out), lambda r: (0, r, 0)),
            pl.BlockSpec((1024, d_out), lambda r: (r, 0)),
            pl.BlockSpec((1024, 1), lambda r: (r, 0)),
            pl.BlockSpec((1, d_out), lambda r: (0, 0)),
        ],
        out_specs=pl.BlockSpec((1024, d_out), lambda r: (r, 0)),
        out_shape=jax.ShapeDtypeStruct((npad, d_out), jnp.float32),
    )(agg2, y2, dis, b2.reshape(1, d_out))

    return out[:n]
